# baseline (speedup 1.0000x reference)
import functools

import jax
import jax.numpy as jnp
from jax import lax
from jax.experimental import pallas as pl
from jax.experimental.pallas import tpu as pltpu

F32 = jnp.float32

N_USER = 60000
N_ITEM = 40000
N_NODES = N_USER + N_ITEM
D = 32
B = 4096
LAYERS = 3
AUX = 10246

KA = 2304
KB_LO, KB_HI = 2304, 10240

BT_FEATS = 256


def _feats_body(aux_ref, wa_ref, wb_ref, ws_ref, out_ref):
    xf = aux_ref[...].astype(F32)
    a = lax.dot_general(xf[:, :KA], wa_ref[...],
                        (((1,), (0,)), ((), ())), preferred_element_type=F32)
    bm = lax.dot_general(xf[:, KB_LO:KB_HI], wb_ref[...],
                         (((1,), (0,)), ((), ())), preferred_element_type=F32)

    x0 = xf[:, 10240:10241]
    x1 = xf[:, 10241:10242]
    xg = xf[:, 10242:10243]
    xa = xf[:, 10243:10244]
    xo = xf[:, 10244:10245]
    xz = xf[:, 10245:10246]

    ws = ws_ref[...]
    rate = a[:, 0:32] + ws[0:1, :]
    genre = a[:, 32:64] / a[:, 128:129]
    direc = a[:, 64:96] / a[:, 129:130]
    act_num = a[:, 96:128] + bm[:, 0:32] + x0 * ws[9:10, :] + x1 * ws[10:11, :]
    act_den = a[:, 130:131] + bm[:, 32:33] + x0 + x1
    actor = act_num / act_den
    gender = ws[1:2, :] + xg * ws[5:6, :]
    age = ws[2:3, :] + xa * ws[6:7, :]
    occ = ws[3:4, :] + xo * ws[7:8, :]
    area = ws[4:5, :] + xz * ws[8:9, :]
    out_ref[...] = jnp.concatenate(
        [rate, genre, direc, actor, gender, age, occ, area], axis=1)


def _feats(aux_info, wa, wb, ws):
    grid = (B // BT_FEATS,)
    return pl.pallas_call(
        _feats_body,
        grid=grid,
        in_specs=[
            pl.BlockSpec((BT_FEATS, AUX), lambda i: (i, 0)),
            pl.BlockSpec((KA, 160), lambda i: (0, 0)),
            pl.BlockSpec((KB_HI - KB_LO, 64), lambda i: (0, 0)),
            pl.BlockSpec((16, 32), lambda i: (0, 0)),
        ],
        out_specs=pl.BlockSpec((BT_FEATS, 256), lambda i: (i, 0)),
        out_shape=jax.ShapeDtypeStruct((B, 256), F32),
    )(aux_info, wa, wb, ws)


def _mlp_body(x_ref, ug_ref, ig_ref, w1_ref, b1_ref, w2_ref, b2_ref,
              w3_ref, b3_ref, out_ref):
    x = x_ref[...]
    w1 = w1_ref[...]
    h = lax.dot_general(x, w1[:, 0:256], (((1,), (1,)), ((), ())),
                        preferred_element_type=F32)
    h += lax.dot_general(ug_ref[...], w1[:, 256:288], (((1,), (1,)), ((), ())),
                         preferred_element_type=F32)
    h += lax.dot_general(ig_ref[...], w1[:, 288:320], (((1,), (1,)), ((), ())),
                         preferred_element_type=F32)
    h = jnp.maximum(h + b1_ref[...], 0.0)
    h = jnp.maximum(
        lax.dot_general(h, w2_ref[...], (((1,), (1,)), ((), ())),
                        preferred_element_type=F32) + b2_ref[...], 0.0)
    out_ref[...] = jnp.sum(h * w3_ref[...], axis=1, keepdims=True) + b3_ref[0, 0]


def _mlp(feats, ug, ig, fc1_W, fc1_b, fc2_W, fc2_b, out_W, out_b):
    w1 = jnp.pad(fc1_W, ((0, 64), (0, 0)))
    b1 = jnp.pad(fc1_b, (0, 64)).reshape(1, 128)
    w2 = jnp.pad(fc2_W, ((0, 64), (0, 64)))
    b2 = jnp.pad(fc2_b, (0, 64)).reshape(1, 128)
    w3 = jnp.pad(out_W, ((0, 0), (0, 64)))
    return pl.pallas_call(
        _mlp_body,
        in_specs=[
            pl.BlockSpec((B, 256), lambda: (0, 0)),
            pl.BlockSpec((B, 32), lambda: (0, 0)),
            pl.BlockSpec((B, 32), lambda: (0, 0)),
            pl.BlockSpec((128, 320), lambda: (0, 0)),
            pl.BlockSpec((1, 128), lambda: (0, 0)),
            pl.BlockSpec((128, 128), lambda: (0, 0)),
            pl.BlockSpec((1, 128), lambda: (0, 0)),
            pl.BlockSpec((1, 128), lambda: (0, 0)),
            pl.BlockSpec(memory_space=pltpu.SMEM),
        ],
        out_specs=pl.BlockSpec((B, 1), lambda: (0, 0)),
        out_shape=jax.ShapeDtypeStruct((B, 1), F32),
    )(feats, ug, ig, w1, b1, w2, b2, w3, out_b.reshape(1, 1))


def kernel(aux_info, user_ids, item_ids, g_src, g_dst, g_vals,
           emb_rate, W_genre, W_director, W_actor,
           emb_gender, emb_age, emb_occ, emb_area,
           emb_user_rel, emb_item_rel,
           fc1_W, fc1_b, fc2_W, fc2_b, out_W, out_b):
    wa = jnp.zeros((KA, 160), F32)
    wa = wa.at[0, 0:32].set(emb_rate[1] - emb_rate[0])
    wa = wa.at[1:26, 32:64].set(W_genre.T)
    wa = wa.at[26:2212, 64:96].set(W_director.T)
    wa = wa.at[2212:KA, 96:128].set(W_actor.T[0:KA - 2212])
    wa = wa.at[1:26, 128].set(1.0)
    wa = wa.at[26:2212, 129].set(1.0)
    wa = wa.at[2212:KA, 130].set(1.0)

    wb = jnp.zeros((KB_HI - KB_LO, 64), F32)
    wb = wb.at[:, 0:32].set(W_actor.T[KB_LO - 2212:KB_HI - 2212])
    wb = wb.at[:, 32].set(1.0)

    ws = jnp.zeros((16, 32), F32)
    ws = ws.at[0].set(emb_rate[0])
    ws = ws.at[1].set(emb_gender[0])
    ws = ws.at[2].set(emb_age[0])
    ws = ws.at[3].set(emb_occ[0])
    ws = ws.at[4].set(emb_area[0])
    ws = ws.at[5].set(emb_gender[1] - emb_gender[0])
    ws = ws.at[6].set(emb_age[1] - emb_age[0])
    ws = ws.at[7].set(emb_occ[1] - emb_occ[0])
    ws = ws.at[8].set(emb_area[1] - emb_area[0])
    ws = ws.at[9].set(W_actor.T[10240 - 2212])
    ws = ws.at[10].set(W_actor.T[10241 - 2212])

    feats = _feats(aux_info, wa, wb, ws)

    all_emb = jnp.concatenate([emb_user_rel, emb_item_rel], axis=0)
    embs = [all_emb]
    for _ in range(LAYERS):
        msg = g_vals[:, None] * jnp.take(all_emb, g_src, axis=0)
        all_emb = jax.ops.segment_sum(msg, g_dst, num_segments=N_NODES)
        embs.append(all_emb)
    light_out = jnp.mean(jnp.stack(embs, axis=1), axis=1)
    ug = jnp.take(light_out[:N_USER], user_ids, axis=0)
    ig = jnp.take(light_out[N_USER:], item_ids, axis=0)

    return _mlp(feats, ug, ig, fc1_W, fc1_b, fc2_W, fc2_b, out_W, out_b)


# speedup vs baseline: 1.6621x; 1.6621x over previous
import functools

import jax
import jax.numpy as jnp
from jax import lax
from jax.experimental import pallas as pl
from jax.experimental.pallas import tpu as pltpu
from jax.experimental.pallas import tpu_sc as plsc

F32 = jnp.float32

N_USER = 60000
N_ITEM = 40000
N_NODES = N_USER + N_ITEM
D = 32
B = 4096
LAYERS = 3
AUX = 10246

KA = 2304
KB_LO, KB_HI = 2304, 10240

BT_FEATS = 256


def _feats_body(aux_ref, wa_ref, wb_ref, ws_ref, out_ref):
    xf = aux_ref[...].astype(F32)
    a = lax.dot_general(xf[:, :KA], wa_ref[...],
                        (((1,), (0,)), ((), ())), preferred_element_type=F32)
    bm = lax.dot_general(xf[:, KB_LO:KB_HI], wb_ref[...],
                         (((1,), (0,)), ((), ())), preferred_element_type=F32)

    x0 = xf[:, 10240:10241]
    x1 = xf[:, 10241:10242]
    xg = xf[:, 10242:10243]
    xa = xf[:, 10243:10244]
    xo = xf[:, 10244:10245]
    xz = xf[:, 10245:10246]

    ws = ws_ref[...]
    rate = a[:, 0:32] + ws[0:1, :]
    genre = a[:, 32:64] / a[:, 128:129]
    direc = a[:, 64:96] / a[:, 129:130]
    act_num = a[:, 96:128] + bm[:, 0:32] + x0 * ws[9:10, :] + x1 * ws[10:11, :]
    act_den = a[:, 130:131] + bm[:, 32:33] + x0 + x1
    actor = act_num / act_den
    gender = ws[1:2, :] + xg * ws[5:6, :]
    age = ws[2:3, :] + xa * ws[6:7, :]
    occ = ws[3:4, :] + xo * ws[7:8, :]
    area = ws[4:5, :] + xz * ws[8:9, :]
    out_ref[...] = jnp.concatenate(
        [rate, genre, direc, actor, gender, age, occ, area], axis=1)


def _feats(aux_info, wa, wb, ws):
    grid = (B // BT_FEATS,)
    return pl.pallas_call(
        _feats_body,
        grid=grid,
        in_specs=[
            pl.BlockSpec((BT_FEATS, AUX), lambda i: (i, 0)),
            pl.BlockSpec((KA, 160), lambda i: (0, 0)),
            pl.BlockSpec((KB_HI - KB_LO, 64), lambda i: (0, 0)),
            pl.BlockSpec((16, 32), lambda i: (0, 0)),
        ],
        out_specs=pl.BlockSpec((BT_FEATS, 256), lambda i: (i, 0)),
        out_shape=jax.ShapeDtypeStruct((B, 256), F32),
    )(aux_info, wa, wb, ws)


def _mlp_body(x_ref, ug_ref, ig_ref, w1_ref, b1_ref, w2_ref, b2_ref,
              w3_ref, b3_ref, out_ref):
    x = x_ref[...]
    w1 = w1_ref[...]
    h = lax.dot_general(x, w1[:, 0:256], (((1,), (1,)), ((), ())),
                        preferred_element_type=F32)
    h += lax.dot_general(ug_ref[...], w1[:, 256:288], (((1,), (1,)), ((), ())),
                         preferred_element_type=F32)
    h += lax.dot_general(ig_ref[...], w1[:, 288:320], (((1,), (1,)), ((), ())),
                         preferred_element_type=F32)
    h = jnp.maximum(h + b1_ref[...], 0.0)
    h = jnp.maximum(
        lax.dot_general(h, w2_ref[...], (((1,), (1,)), ((), ())),
                        preferred_element_type=F32) + b2_ref[...], 0.0)
    out_ref[...] = jnp.sum(h * w3_ref[...], axis=1, keepdims=True) + b3_ref[0, 0]


def _mlp(feats, ug, ig, fc1_W, fc1_b, fc2_W, fc2_b, out_W, out_b):
    w1 = jnp.pad(fc1_W, ((0, 64), (0, 0)))
    b1 = jnp.pad(fc1_b, (0, 64)).reshape(1, 128)
    w2 = jnp.pad(fc2_W, ((0, 64), (0, 64)))
    b2 = jnp.pad(fc2_b, (0, 64)).reshape(1, 128)
    w3 = jnp.pad(out_W, ((0, 0), (0, 64)))
    return pl.pallas_call(
        _mlp_body,
        in_specs=[
            pl.BlockSpec((B, 256), lambda: (0, 0)),
            pl.BlockSpec((B, 32), lambda: (0, 0)),
            pl.BlockSpec((B, 32), lambda: (0, 0)),
            pl.BlockSpec((128, 320), lambda: (0, 0)),
            pl.BlockSpec((1, 128), lambda: (0, 0)),
            pl.BlockSpec((128, 128), lambda: (0, 0)),
            pl.BlockSpec((1, 128), lambda: (0, 0)),
            pl.BlockSpec((1, 128), lambda: (0, 0)),
            pl.BlockSpec(memory_space=pltpu.SMEM),
        ],
        out_specs=pl.BlockSpec((B, 1), lambda: (0, 0)),
        out_shape=jax.ShapeDtypeStruct((B, 1), F32),
    )(feats, ug, ig, w1, b1, w2, b2, w3, out_b.reshape(1, 1))


NC = 2
NS = 16
HALF = N_NODES // 2
HALF_PAD = 50176
TRASH = HALF
EP = 1_638_400
EROWS = EP // 128
ROWS_PER_TILE = EROWS // NS
GROUPS = ROWS_PER_TILE // 4

_mesh = plsc.VectorSubcoreMesh(core_axis_name="c", subcore_axis_name="s")


def _gcn_layer_body(table, src2, dst1, vals1, out,
                    acc, srcb, ldstb, dstb1, valsb1, rows, sem):
    c = lax.axis_index("c")
    s = lax.axis_index("s")
    _iota16 = lax.iota(jnp.int32, 16)

    def _zrow(r, _):
        rows[r, 0:16] = jnp.zeros((16,), F32)
        rows[r, 16:32] = jnp.zeros((16,), F32)
        return _
    lax.fori_loop(0, 392, _zrow, None)
    zb = s * 3136

    def _zcp(i, _):
        pltpu.sync_copy(rows.at[pl.ds(0, 392)],
                        acc.at[pl.ds(zb + i * 392, 392)])
        return _
    lax.fori_loop(0, 8, _zcp, None)
    plsc.subcore_barrier()

    lo = c * HALF

    def _group(g, _):
        rb = s * ROWS_PER_TILE + g * 4
        eb = rb * 128
        pltpu.sync_copy(src2.at[pl.ds(rb, 4)], srcb)
        pltpu.sync_copy(dst1.at[pl.ds(eb, 512)], dstb1)
        pltpu.sync_copy(vals1.at[pl.ds(eb, 512)], valsb1)
        cps = [pltpu.async_copy(table.at[srcb.at[j]],
                                rows.at[pl.ds(j * 128, 128)], sem)
               for j in range(4)]
        for cp in cps:
            cp.wait()

        def _chunk(k, _):
            e16 = k * 16 + _iota16
            v = plsc.load_gather(valsb1, [e16])
            d = plsc.load_gather(dstb1, [e16])
            locd = d - lo
            ok = (locd >= 0) & (locd < HALF)
            ld = jnp.where(ok, locd, TRASH)
            plsc.store_scatter(
                ldstb, [lax.broadcast(k // 8, (16,)),
                        (k % 8) * 16 + _iota16], ld)
            for col in range(D):
                cvec = jnp.full((16,), col, jnp.int32)
                x = plsc.load_gather(rows, [e16, cvec])
                plsc.store_scatter(rows, [e16, cvec], x * v)
            return _
        lax.fori_loop(0, 32, _chunk, None)

        for j in range(4):
            pltpu.sync_copy(rows.at[pl.ds(j * 128, 128)],
                            acc.at[ldstb.at[j]], add=True)
        return _
    lax.fori_loop(0, GROUPS, _group, None)

    plsc.subcore_barrier()
    ob = s * 3128

    @pl.when(s < 15)
    def _():
        pltpu.sync_copy(acc.at[pl.ds(ob, 3128)],
                        out.at[pl.ds(lo + ob, 3128)])

    @pl.when(s == 15)
    def _():
        pltpu.sync_copy(acc.at[pl.ds(ob, 3080)],
                        out.at[pl.ds(lo + ob, 3080)])


def _gcn_layer(table, src2, dst1, vals1):
    return pl.kernel(
        _gcn_layer_body,
        out_type=jax.ShapeDtypeStruct((N_NODES, D), F32),
        mesh=_mesh,
        compiler_params=pltpu.CompilerParams(needs_layout_passes=False, use_tc_tiling_on_sc=False),
        scratch_types=[
            pltpu.VMEM_SHARED((HALF_PAD, D), F32),
            pltpu.VMEM((4, 128), jnp.int32),
            pltpu.VMEM((4, 128), jnp.int32),
            pltpu.VMEM((512,), jnp.int32),
            pltpu.VMEM((512,), F32),
            pltpu.VMEM((512, D), F32),
            pltpu.SemaphoreType.DMA,
        ],
    )(table, src2, dst1, vals1)


def _gather_mean_body(t0, t1, t2, t3, uid1, iid1, ug, ig,
                      idxb, rowsb, accb, sem):
    c = lax.axis_index("c")
    s = lax.axis_index("s")
    w = s * NC + c

    def _accum(table, first):
        if first:
            pltpu.async_copy(table.at[idxb], accb, sem).wait()
        else:
            pltpu.async_copy(table.at[idxb], rowsb, sem).wait()

            def _add(r, _):
                accb[r, 0:16] = accb[r, 0:16] + rowsb[r, 0:16]
                accb[r, 16:32] = accb[r, 16:32] + rowsb[r, 16:32]
                return _
            lax.fori_loop(0, 128, _add, None)

    def _quarter(r, _):
        accb[r, 0:16] = accb[r, 0:16] * 0.25
        accb[r, 16:32] = accb[r, 16:32] * 0.25
        return _

    pltpu.sync_copy(uid1.at[pl.ds(w * 128, 128)], idxb)
    for i, t in enumerate((t0, t1, t2, t3)):
        _accum(t, i == 0)
    lax.fori_loop(0, 128, _quarter, None)
    pltpu.sync_copy(accb, ug.at[pl.ds(w * 128, 128)])

    pltpu.sync_copy(iid1.at[pl.ds(w * 128, 128)], idxb)
    for kk in range(8):
        sl = pl.ds(kk * 16, 16)
        idxb[sl] = idxb[sl] + N_USER
    for i, t in enumerate((t0, t1, t2, t3)):
        _accum(t, i == 0)
    lax.fori_loop(0, 128, _quarter, None)
    pltpu.sync_copy(accb, ig.at[pl.ds(w * 128, 128)])


def _gather_mean(t0, t1, t2, t3, uid1, iid1):
    return pl.kernel(
        _gather_mean_body,
        out_type=(jax.ShapeDtypeStruct((B, D), F32),
                  jax.ShapeDtypeStruct((B, D), F32)),
        mesh=_mesh,
        compiler_params=pltpu.CompilerParams(needs_layout_passes=False, use_tc_tiling_on_sc=False),
        scratch_types=[
            pltpu.VMEM((128,), jnp.int32),
            pltpu.VMEM((128, D), F32),
            pltpu.VMEM((128, D), F32),
            pltpu.SemaphoreType.DMA,
        ],
    )(t0, t1, t2, t3, uid1, iid1)


def kernel(aux_info, user_ids, item_ids, g_src, g_dst, g_vals,
           emb_rate, W_genre, W_director, W_actor,
           emb_gender, emb_age, emb_occ, emb_area,
           emb_user_rel, emb_item_rel,
           fc1_W, fc1_b, fc2_W, fc2_b, out_W, out_b):
    wa = jnp.zeros((KA, 160), F32)
    wa = wa.at[0, 0:32].set(emb_rate[1] - emb_rate[0])
    wa = wa.at[1:26, 32:64].set(W_genre.T)
    wa = wa.at[26:2212, 64:96].set(W_director.T)
    wa = wa.at[2212:KA, 96:128].set(W_actor.T[0:KA - 2212])
    wa = wa.at[1:26, 128].set(1.0)
    wa = wa.at[26:2212, 129].set(1.0)
    wa = wa.at[2212:KA, 130].set(1.0)

    wb = jnp.zeros((KB_HI - KB_LO, 64), F32)
    wb = wb.at[:, 0:32].set(W_actor.T[KB_LO - 2212:KB_HI - 2212])
    wb = wb.at[:, 32].set(1.0)

    ws = jnp.zeros((16, 32), F32)
    ws = ws.at[0].set(emb_rate[0])
    ws = ws.at[1].set(emb_gender[0])
    ws = ws.at[2].set(emb_age[0])
    ws = ws.at[3].set(emb_occ[0])
    ws = ws.at[4].set(emb_area[0])
    ws = ws.at[5].set(emb_gender[1] - emb_gender[0])
    ws = ws.at[6].set(emb_age[1] - emb_age[0])
    ws = ws.at[7].set(emb_occ[1] - emb_occ[0])
    ws = ws.at[8].set(emb_area[1] - emb_area[0])
    ws = ws.at[9].set(W_actor.T[10240 - 2212])
    ws = ws.at[10].set(W_actor.T[10241 - 2212])

    feats = _feats(aux_info, wa, wb, ws)

    pad = EP - g_src.shape[0]
    src2 = jnp.pad(g_src, (0, pad)).reshape(EROWS, 128)
    dst1 = jnp.pad(g_dst, (0, pad))
    vals1 = jnp.pad(g_vals, (0, pad))

    t0 = jnp.concatenate([emb_user_rel, emb_item_rel], axis=0)
    t1 = _gcn_layer(t0, src2, dst1, vals1)
    t2 = _gcn_layer(t1, src2, dst1, vals1)
    t3 = _gcn_layer(t2, src2, dst1, vals1)

    ug, ig = _gather_mean(t0, t1, t2, t3, user_ids, item_ids)

    return _mlp(feats, ug, ig, fc1_W, fc1_b, fc2_W, fc2_b, out_W, out_b)


# speedup vs baseline: 2.0897x; 1.2573x over previous
import functools

import jax
import jax.numpy as jnp
from jax import lax
from jax.experimental import pallas as pl
from jax.experimental.pallas import tpu as pltpu
from jax.experimental.pallas import tpu_sc as plsc

F32 = jnp.float32

N_USER = 60000
N_ITEM = 40000
N_NODES = N_USER + N_ITEM
D = 32
B = 4096
LAYERS = 3
AUX = 10246

KA = 2304
KB_LO, KB_HI = 2304, 10240

BT_FEATS = 256


def _feats_body(aux_ref, wa_ref, wb_ref, ws_ref, out_ref):
    xf = aux_ref[...].astype(F32)
    a = lax.dot_general(xf[:, :KA], wa_ref[...],
                        (((1,), (0,)), ((), ())), preferred_element_type=F32)
    bm = lax.dot_general(xf[:, KB_LO:KB_HI], wb_ref[...],
                         (((1,), (0,)), ((), ())), preferred_element_type=F32)

    x0 = xf[:, 10240:10241]
    x1 = xf[:, 10241:10242]
    xg = xf[:, 10242:10243]
    xa = xf[:, 10243:10244]
    xo = xf[:, 10244:10245]
    xz = xf[:, 10245:10246]

    ws = ws_ref[...]
    rate = a[:, 0:32] + ws[0:1, :]
    genre = a[:, 32:64] / a[:, 128:129]
    direc = a[:, 64:96] / a[:, 129:130]
    act_num = a[:, 96:128] + bm[:, 0:32] + x0 * ws[9:10, :] + x1 * ws[10:11, :]
    act_den = a[:, 130:131] + bm[:, 32:33] + x0 + x1
    actor = act_num / act_den
    gender = ws[1:2, :] + xg * ws[5:6, :]
    age = ws[2:3, :] + xa * ws[6:7, :]
    occ = ws[3:4, :] + xo * ws[7:8, :]
    area = ws[4:5, :] + xz * ws[8:9, :]
    out_ref[...] = jnp.concatenate(
        [rate, genre, direc, actor, gender, age, occ, area], axis=1)


def _feats(aux_info, wa, wb, ws):
    grid = (B // BT_FEATS,)
    return pl.pallas_call(
        _feats_body,
        grid=grid,
        in_specs=[
            pl.BlockSpec((BT_FEATS, AUX), lambda i: (i, 0)),
            pl.BlockSpec((KA, 160), lambda i: (0, 0)),
            pl.BlockSpec((KB_HI - KB_LO, 64), lambda i: (0, 0)),
            pl.BlockSpec((16, 32), lambda i: (0, 0)),
        ],
        out_specs=pl.BlockSpec((BT_FEATS, 256), lambda i: (i, 0)),
        out_shape=jax.ShapeDtypeStruct((B, 256), F32),
    )(aux_info, wa, wb, ws)


def _mlp_body(x_ref, ug_ref, ig_ref, w1_ref, b1_ref, w2_ref, b2_ref,
              w3_ref, b3_ref, out_ref):
    x = x_ref[...]
    w1 = w1_ref[...]
    h = lax.dot_general(x, w1[:, 0:256], (((1,), (1,)), ((), ())),
                        preferred_element_type=F32)
    h += lax.dot_general(ug_ref[...], w1[:, 256:288], (((1,), (1,)), ((), ())),
                         preferred_element_type=F32)
    h += lax.dot_general(ig_ref[...], w1[:, 288:320], (((1,), (1,)), ((), ())),
                         preferred_element_type=F32)
    h = jnp.maximum(h + b1_ref[...], 0.0)
    h = jnp.maximum(
        lax.dot_general(h, w2_ref[...], (((1,), (1,)), ((), ())),
                        preferred_element_type=F32) + b2_ref[...], 0.0)
    out_ref[...] = jnp.sum(h * w3_ref[...], axis=1, keepdims=True) + b3_ref[0, 0]


def _mlp(feats, ug, ig, fc1_W, fc1_b, fc2_W, fc2_b, out_W, out_b):
    w1 = jnp.pad(fc1_W, ((0, 64), (0, 0)))
    b1 = jnp.pad(fc1_b, (0, 64)).reshape(1, 128)
    w2 = jnp.pad(fc2_W, ((0, 64), (0, 64)))
    b2 = jnp.pad(fc2_b, (0, 64)).reshape(1, 128)
    w3 = jnp.pad(out_W, ((0, 0), (0, 64)))
    return pl.pallas_call(
        _mlp_body,
        in_specs=[
            pl.BlockSpec((B, 256), lambda: (0, 0)),
            pl.BlockSpec((B, 32), lambda: (0, 0)),
            pl.BlockSpec((B, 32), lambda: (0, 0)),
            pl.BlockSpec((128, 320), lambda: (0, 0)),
            pl.BlockSpec((1, 128), lambda: (0, 0)),
            pl.BlockSpec((128, 128), lambda: (0, 0)),
            pl.BlockSpec((1, 128), lambda: (0, 0)),
            pl.BlockSpec((1, 128), lambda: (0, 0)),
            pl.BlockSpec(memory_space=pltpu.SMEM),
        ],
        out_specs=pl.BlockSpec((B, 1), lambda: (0, 0)),
        out_shape=jax.ShapeDtypeStruct((B, 1), F32),
    )(feats, ug, ig, w1, b1, w2, b2, w3, out_b.reshape(1, 1))


NC = 2
NS = 16
HALF = N_NODES // 2
HALF_PAD = 50176
TRASH = HALF
EP = 1_638_400
NG = 800

_mesh = plsc.VectorSubcoreMesh(core_axis_name="c", subcore_axis_name="s")


def _gcn_layer_body(table, src1, dst1, vals1, out,
                    acc, msrc, mdst, mvals, ldstb, rows,
                    msem, gsem, ssem):
    c = lax.axis_index("c")
    s = lax.axis_index("s")
    iota = lax.iota(jnp.int32, 16)

    def _zrow(r, _):
        rows[r, 0:16] = jnp.zeros((16,), F32)
        rows[r, 16:32] = jnp.zeros((16,), F32)
        return _
    lax.fori_loop(0, 392, _zrow, None)
    zb = s * 3136

    def _zcp(i, _):
        pltpu.sync_copy(rows.at[pl.ds(0, 392)],
                        acc.at[pl.ds(zb + i * 392, 392)])
        return _
    lax.fori_loop(0, 8, _zcp, None)
    plsc.subcore_barrier()

    lo = c * HALF
    ebase = s * (NG * 128)

    def fire_meta(t):
        m = lax.rem(t, 8)
        eb = ebase + t * 128
        pltpu.async_copy(src1.at[pl.ds(eb, 128)], msrc.at[m], msem.at[m])
        pltpu.async_copy(dst1.at[pl.ds(eb, 128)], mdst.at[m], msem.at[m])
        pltpu.async_copy(vals1.at[pl.ds(eb, 128)], mvals.at[m], msem.at[m])

    def drain_meta(t):
        m = lax.rem(t, 8)
        eb = ebase + t * 128
        pltpu.make_async_copy(src1.at[pl.ds(eb, 128)], msrc.at[m],
                              msem.at[m]).wait()
        pltpu.make_async_copy(dst1.at[pl.ds(eb, 128)], mdst.at[m],
                              msem.at[m]).wait()
        pltpu.make_async_copy(vals1.at[pl.ds(eb, 128)], mvals.at[m],
                              msem.at[m]).wait()

    def fire_gather(t):
        m = lax.rem(t, 8)
        g = lax.rem(t, 6)
        pltpu.async_copy(table.at[msrc.at[m]],
                         rows.at[pl.ds(g * 128, 128)], gsem.at[g])

    def drain_gather(t):
        m = lax.rem(t, 8)
        g = lax.rem(t, 6)
        pltpu.make_async_copy(table.at[msrc.at[m]],
                              rows.at[pl.ds(g * 128, 128)],
                              gsem.at[g]).wait()

    def fire_scatter(t):
        g = lax.rem(t, 6)
        pltpu.async_copy(rows.at[pl.ds(g * 128, 128)],
                         acc.at[ldstb.at[g]], ssem.at[g], add=True)

    def drain_scatter(t):
        g = lax.rem(t, 6)
        pltpu.make_async_copy(rows.at[pl.ds(g * 128, 128)],
                              acc.at[ldstb.at[g]], ssem.at[g]).wait()

    def compute(i):
        m = lax.rem(i, 8)
        g = lax.rem(i, 6)
        bm = lax.broadcast(m, (16,))
        bg = lax.broadcast(g, (16,))
        rbase = g * 128
        for k in range(8):
            col16 = k * 16 + iota
            v = plsc.load_gather(mvals, [bm, col16])
            d = plsc.load_gather(mdst, [bm, col16])
            locd = d - lo
            ok = (locd >= 0) & (locd < HALF)
            ld = jnp.where(ok, locd, TRASH)
            plsc.store_scatter(ldstb, [bg, col16], ld)
            e16 = rbase + col16
            for col in range(D):
                cvec = jnp.full((16,), col, jnp.int32)
                x = plsc.load_gather(rows, [e16, cvec])
                plsc.store_scatter(rows, [e16, cvec], x * v)

    for t in range(7):
        fire_meta(t)
    for t in range(3):
        drain_meta(t)
        fire_gather(t)

    def _iter(i, _):
        @pl.when(i + 7 < NG)
        def _():
            fire_meta(i + 7)

        @pl.when(i >= 3)
        def _():
            drain_scatter(i - 3)

        @pl.when(i + 3 < NG)
        def _():
            drain_meta(i + 3)
            fire_gather(i + 3)

        drain_gather(i)
        compute(i)
        fire_scatter(i)
        return _
    lax.fori_loop(0, NG, _iter, None)

    for t in (NG - 3, NG - 2, NG - 1):
        drain_scatter(t)

    plsc.subcore_barrier()
    ob = s * 3128

    @pl.when(s < 15)
    def _():
        pltpu.sync_copy(acc.at[pl.ds(ob, 3128)],
                        out.at[pl.ds(lo + ob, 3128)])

    @pl.when(s == 15)
    def _():
        pltpu.sync_copy(acc.at[pl.ds(ob, 3080)],
                        out.at[pl.ds(lo + ob, 3080)])


def _gcn_layer(table, src1, dst1, vals1):
    return pl.kernel(
        _gcn_layer_body,
        out_type=jax.ShapeDtypeStruct((N_NODES, D), F32),
        mesh=_mesh,
        compiler_params=pltpu.CompilerParams(
            needs_layout_passes=False, use_tc_tiling_on_sc=False),
        scratch_types=[
            pltpu.VMEM_SHARED((HALF_PAD, D), F32),
            pltpu.VMEM((8, 128), jnp.int32),
            pltpu.VMEM((8, 128), jnp.int32),
            pltpu.VMEM((8, 128), F32),
            pltpu.VMEM((6, 128), jnp.int32),
            pltpu.VMEM((768, D), F32),
            pltpu.SemaphoreType.DMA((8,)),
            pltpu.SemaphoreType.DMA((6,)),
            pltpu.SemaphoreType.DMA((6,)),
        ],
    )(table, src1, dst1, vals1)


def _gather_mean_body(t0, t1, t2, t3, uid1, iid1, ug, ig,
                      idxb, rowsb, accb, sem):
    c = lax.axis_index("c")
    s = lax.axis_index("s")
    w = s * NC + c

    def _accum(table, first):
        if first:
            pltpu.async_copy(table.at[idxb], accb, sem).wait()
        else:
            pltpu.async_copy(table.at[idxb], rowsb, sem).wait()

            def _add(r, _):
                accb[r, 0:16] = accb[r, 0:16] + rowsb[r, 0:16]
                accb[r, 16:32] = accb[r, 16:32] + rowsb[r, 16:32]
                return _
            lax.fori_loop(0, 128, _add, None)

    def _quarter(r, _):
        accb[r, 0:16] = accb[r, 0:16] * 0.25
        accb[r, 16:32] = accb[r, 16:32] * 0.25
        return _

    pltpu.sync_copy(uid1.at[pl.ds(w * 128, 128)], idxb)
    for i, t in enumerate((t0, t1, t2, t3)):
        _accum(t, i == 0)
    lax.fori_loop(0, 128, _quarter, None)
    pltpu.sync_copy(accb, ug.at[pl.ds(w * 128, 128)])

    pltpu.sync_copy(iid1.at[pl.ds(w * 128, 128)], idxb)
    for kk in range(8):
        sl = pl.ds(kk * 16, 16)
        idxb[sl] = idxb[sl] + N_USER
    for i, t in enumerate((t0, t1, t2, t3)):
        _accum(t, i == 0)
    lax.fori_loop(0, 128, _quarter, None)
    pltpu.sync_copy(accb, ig.at[pl.ds(w * 128, 128)])


def _gather_mean(t0, t1, t2, t3, uid1, iid1):
    return pl.kernel(
        _gather_mean_body,
        out_type=(jax.ShapeDtypeStruct((B, D), F32),
                  jax.ShapeDtypeStruct((B, D), F32)),
        mesh=_mesh,
        compiler_params=pltpu.CompilerParams(needs_layout_passes=False, use_tc_tiling_on_sc=False),
        scratch_types=[
            pltpu.VMEM((128,), jnp.int32),
            pltpu.VMEM((128, D), F32),
            pltpu.VMEM((128, D), F32),
            pltpu.SemaphoreType.DMA,
        ],
    )(t0, t1, t2, t3, uid1, iid1)


def kernel(aux_info, user_ids, item_ids, g_src, g_dst, g_vals,
           emb_rate, W_genre, W_director, W_actor,
           emb_gender, emb_age, emb_occ, emb_area,
           emb_user_rel, emb_item_rel,
           fc1_W, fc1_b, fc2_W, fc2_b, out_W, out_b):
    wa = jnp.zeros((KA, 160), F32)
    wa = wa.at[0, 0:32].set(emb_rate[1] - emb_rate[0])
    wa = wa.at[1:26, 32:64].set(W_genre.T)
    wa = wa.at[26:2212, 64:96].set(W_director.T)
    wa = wa.at[2212:KA, 96:128].set(W_actor.T[0:KA - 2212])
    wa = wa.at[1:26, 128].set(1.0)
    wa = wa.at[26:2212, 129].set(1.0)
    wa = wa.at[2212:KA, 130].set(1.0)

    wb = jnp.zeros((KB_HI - KB_LO, 64), F32)
    wb = wb.at[:, 0:32].set(W_actor.T[KB_LO - 2212:KB_HI - 2212])
    wb = wb.at[:, 32].set(1.0)

    ws = jnp.zeros((16, 32), F32)
    ws = ws.at[0].set(emb_rate[0])
    ws = ws.at[1].set(emb_gender[0])
    ws = ws.at[2].set(emb_age[0])
    ws = ws.at[3].set(emb_occ[0])
    ws = ws.at[4].set(emb_area[0])
    ws = ws.at[5].set(emb_gender[1] - emb_gender[0])
    ws = ws.at[6].set(emb_age[1] - emb_age[0])
    ws = ws.at[7].set(emb_occ[1] - emb_occ[0])
    ws = ws.at[8].set(emb_area[1] - emb_area[0])
    ws = ws.at[9].set(W_actor.T[10240 - 2212])
    ws = ws.at[10].set(W_actor.T[10241 - 2212])

    feats = _feats(aux_info, wa, wb, ws)

    pad = EP - g_src.shape[0]
    src1 = jnp.pad(g_src, (0, pad))
    dst1 = jnp.pad(g_dst, (0, pad))
    vals1 = jnp.pad(g_vals, (0, pad))

    t0 = jnp.concatenate([emb_user_rel, emb_item_rel], axis=0)
    t1 = _gcn_layer(t0, src1, dst1, vals1)
    t2 = _gcn_layer(t1, src1, dst1, vals1)
    t3 = _gcn_layer(t2, src1, dst1, vals1)

    ug, ig = _gather_mean(t0, t1, t2, t3, user_ids, item_ids)

    return _mlp(feats, ug, ig, fc1_W, fc1_b, fc2_W, fc2_b, out_W, out_b)


# speedup vs baseline: 5.6109x; 2.6851x over previous
import functools

import jax
import jax.numpy as jnp
from jax import lax
from jax.experimental import pallas as pl
from jax.experimental.pallas import tpu as pltpu
from jax.experimental.pallas import tpu_sc as plsc

F32 = jnp.float32

N_USER = 60000
N_ITEM = 40000
N_NODES = N_USER + N_ITEM
D = 32
B = 4096
LAYERS = 3
AUX = 10246

KA = 2304
KB_LO, KB_HI = 2304, 10240

BT_FEATS = 256


def _feats_body(aux_ref, wa_ref, wb_ref, ws_ref, out_ref):
    xf = aux_ref[...].astype(F32)
    a = lax.dot_general(xf[:, :KA], wa_ref[...],
                        (((1,), (0,)), ((), ())), preferred_element_type=F32)
    bm = lax.dot_general(xf[:, KB_LO:KB_HI], wb_ref[...],
                         (((1,), (0,)), ((), ())), preferred_element_type=F32)

    x0 = xf[:, 10240:10241]
    x1 = xf[:, 10241:10242]
    xg = xf[:, 10242:10243]
    xa = xf[:, 10243:10244]
    xo = xf[:, 10244:10245]
    xz = xf[:, 10245:10246]

    ws = ws_ref[...]
    rate = a[:, 0:32] + ws[0:1, :]
    genre = a[:, 32:64] / a[:, 128:129]
    direc = a[:, 64:96] / a[:, 129:130]
    act_num = a[:, 96:128] + bm[:, 0:32] + x0 * ws[9:10, :] + x1 * ws[10:11, :]
    act_den = a[:, 130:131] + bm[:, 32:33] + x0 + x1
    actor = act_num / act_den
    gender = ws[1:2, :] + xg * ws[5:6, :]
    age = ws[2:3, :] + xa * ws[6:7, :]
    occ = ws[3:4, :] + xo * ws[7:8, :]
    area = ws[4:5, :] + xz * ws[8:9, :]
    out_ref[...] = jnp.concatenate(
        [rate, genre, direc, actor, gender, age, occ, area], axis=1)


def _feats(aux_info, wa, wb, ws):
    grid = (B // BT_FEATS,)
    return pl.pallas_call(
        _feats_body,
        grid=grid,
        in_specs=[
            pl.BlockSpec((BT_FEATS, AUX), lambda i: (i, 0)),
            pl.BlockSpec((KA, 160), lambda i: (0, 0)),
            pl.BlockSpec((KB_HI - KB_LO, 64), lambda i: (0, 0)),
            pl.BlockSpec((16, 32), lambda i: (0, 0)),
        ],
        out_specs=pl.BlockSpec((BT_FEATS, 256), lambda i: (i, 0)),
        out_shape=jax.ShapeDtypeStruct((B, 256), F32),
    )(aux_info, wa, wb, ws)


def _mlp_body(x_ref, ug_ref, ig_ref, w1_ref, b1_ref, w2_ref, b2_ref,
              w3_ref, b3_ref, out_ref):
    x = x_ref[...]
    w1 = w1_ref[...]
    h = lax.dot_general(x, w1[:, 0:256], (((1,), (1,)), ((), ())),
                        preferred_element_type=F32)
    h += lax.dot_general(ug_ref[...], w1[:, 256:288], (((1,), (1,)), ((), ())),
                         preferred_element_type=F32)
    h += lax.dot_general(ig_ref[...], w1[:, 288:320], (((1,), (1,)), ((), ())),
                         preferred_element_type=F32)
    h = jnp.maximum(h + b1_ref[...], 0.0)
    h = jnp.maximum(
        lax.dot_general(h, w2_ref[...], (((1,), (1,)), ((), ())),
                        preferred_element_type=F32) + b2_ref[...], 0.0)
    out_ref[...] = jnp.sum(h * w3_ref[...], axis=1, keepdims=True) + b3_ref[0, 0]


def _mlp(feats, ug, ig, fc1_W, fc1_b, fc2_W, fc2_b, out_W, out_b):
    w1 = jnp.pad(fc1_W, ((0, 64), (0, 0)))
    b1 = jnp.pad(fc1_b, (0, 64)).reshape(1, 128)
    w2 = jnp.pad(fc2_W, ((0, 64), (0, 64)))
    b2 = jnp.pad(fc2_b, (0, 64)).reshape(1, 128)
    w3 = jnp.pad(out_W, ((0, 0), (0, 64)))
    return pl.pallas_call(
        _mlp_body,
        in_specs=[
            pl.BlockSpec((B, 256), lambda: (0, 0)),
            pl.BlockSpec((B, 32), lambda: (0, 0)),
            pl.BlockSpec((B, 32), lambda: (0, 0)),
            pl.BlockSpec((128, 320), lambda: (0, 0)),
            pl.BlockSpec((1, 128), lambda: (0, 0)),
            pl.BlockSpec((128, 128), lambda: (0, 0)),
            pl.BlockSpec((1, 128), lambda: (0, 0)),
            pl.BlockSpec((1, 128), lambda: (0, 0)),
            pl.BlockSpec(memory_space=pltpu.SMEM),
        ],
        out_specs=pl.BlockSpec((B, 1), lambda: (0, 0)),
        out_shape=jax.ShapeDtypeStruct((B, 1), F32),
    )(feats, ug, ig, w1, b1, w2, b2, w3, out_b.reshape(1, 1))


NC = 2
NS = 16
HALF = N_NODES // 2
HALF_PAD = 50176
TRASH = HALF
EP = 1_638_400
NG = 800

_mesh = plsc.VectorSubcoreMesh(core_axis_name="c", subcore_axis_name="s")


def _gcn_layer_body(table, src1, dst1, vals1, out,
                    acc, msrc, mdst, mvals, ldstb, rows,
                    msem, gsem, ssem):
    c = lax.axis_index("c")
    s = lax.axis_index("s")
    iota = lax.iota(jnp.int32, 16)

    def _zrow(r, _):
        rows[r, 0:16] = jnp.zeros((16,), F32)
        rows[r, 16:32] = jnp.zeros((16,), F32)
        return _
    lax.fori_loop(0, 392, _zrow, None)
    zb = s * 3136

    def _zcp(i, _):
        pltpu.sync_copy(rows.at[pl.ds(0, 392)],
                        acc.at[pl.ds(zb + i * 392, 392)])
        return _
    lax.fori_loop(0, 8, _zcp, None)
    plsc.subcore_barrier()

    lo = c * HALF
    ebase = s * (NG * 128)

    def fire_meta(t):
        m = lax.rem(t, 8)
        eb = ebase + t * 128
        pltpu.async_copy(src1.at[pl.ds(eb, 128)], msrc.at[m], msem.at[m])
        pltpu.async_copy(dst1.at[pl.ds(eb, 128)], mdst.at[m], msem.at[m])
        pltpu.async_copy(vals1.at[pl.ds(eb, 128)], mvals.at[m], msem.at[m])

    def drain_meta(t):
        m = lax.rem(t, 8)
        eb = ebase + t * 128
        pltpu.make_async_copy(src1.at[pl.ds(eb, 128)], msrc.at[m],
                              msem.at[m]).wait()
        pltpu.make_async_copy(dst1.at[pl.ds(eb, 128)], mdst.at[m],
                              msem.at[m]).wait()
        pltpu.make_async_copy(vals1.at[pl.ds(eb, 128)], mvals.at[m],
                              msem.at[m]).wait()

    def fire_gather(t):
        m = lax.rem(t, 8)
        g = lax.rem(t, 6)
        pltpu.async_copy(table.at[msrc.at[m]],
                         rows.at[pl.ds(g * 128, 128)], gsem.at[g])

    def drain_gather(t):
        m = lax.rem(t, 8)
        g = lax.rem(t, 6)
        pltpu.make_async_copy(table.at[msrc.at[m]],
                              rows.at[pl.ds(g * 128, 128)],
                              gsem.at[g]).wait()

    def fire_scatter(t):
        g = lax.rem(t, 6)
        pltpu.async_copy(rows.at[pl.ds(g * 128, 128)],
                         acc.at[ldstb.at[g]], ssem.at[g], add=True)

    def drain_scatter(t):
        g = lax.rem(t, 6)
        pltpu.make_async_copy(rows.at[pl.ds(g * 128, 128)],
                              acc.at[ldstb.at[g]], ssem.at[g]).wait()

    def compute(i):
        m = lax.rem(i, 8)
        g = lax.rem(i, 6)
        rbase = g * 128
        for k in range(8):
            v16 = mvals[m, pl.ds(k * 16, 16)]
            d = mdst[m, pl.ds(k * 16, 16)]
            locd = d - lo
            ok = (locd >= 0) & (locd < HALF)
            ld = jnp.where(ok, locd, TRASH)
            ldstb[g, pl.ds(k * 16, 16)] = ld
            for j in range(16):
                bv = lax.gather(
                    v16, jnp.full((16, 1), j, jnp.int32),
                    lax.GatherDimensionNumbers(
                        offset_dims=(), collapsed_slice_dims=(0,),
                        start_index_map=(0,)),
                    (1,), mode=lax.GatherScatterMode.PROMISE_IN_BOUNDS)
                e = rbase + k * 16 + j
                rows[e, 0:16] = rows[e, 0:16] * bv
                rows[e, 16:32] = rows[e, 16:32] * bv

    for t in range(7):
        fire_meta(t)
    for t in range(3):
        drain_meta(t)
        fire_gather(t)

    def _iter(i, _):
        @pl.when(i + 7 < NG)
        def _():
            fire_meta(i + 7)

        @pl.when(i >= 3)
        def _():
            drain_scatter(i - 3)

        @pl.when(i + 3 < NG)
        def _():
            drain_meta(i + 3)
            fire_gather(i + 3)

        drain_gather(i)
        compute(i)
        fire_scatter(i)
        return _
    lax.fori_loop(0, NG, _iter, None)

    for t in (NG - 3, NG - 2, NG - 1):
        drain_scatter(t)

    plsc.subcore_barrier()
    ob = s * 3128

    @pl.when(s < 15)
    def _():
        pltpu.sync_copy(acc.at[pl.ds(ob, 3128)],
                        out.at[pl.ds(lo + ob, 3128)])

    @pl.when(s == 15)
    def _():
        pltpu.sync_copy(acc.at[pl.ds(ob, 3080)],
                        out.at[pl.ds(lo + ob, 3080)])


def _gcn_layer(table, src1, dst1, vals1):
    return pl.kernel(
        _gcn_layer_body,
        out_type=jax.ShapeDtypeStruct((N_NODES, D), F32),
        mesh=_mesh,
        compiler_params=pltpu.CompilerParams(
            needs_layout_passes=False, use_tc_tiling_on_sc=False),
        scratch_types=[
            pltpu.VMEM_SHARED((HALF_PAD, D), F32),
            pltpu.VMEM((8, 128), jnp.int32),
            pltpu.VMEM((8, 128), jnp.int32),
            pltpu.VMEM((8, 128), F32),
            pltpu.VMEM((6, 128), jnp.int32),
            pltpu.VMEM((768, D), F32),
            pltpu.SemaphoreType.DMA((8,)),
            pltpu.SemaphoreType.DMA((6,)),
            pltpu.SemaphoreType.DMA((6,)),
        ],
    )(table, src1, dst1, vals1)


def _gather_mean_body(t0, t1, t2, t3, uid1, iid1, ug, ig,
                      idxb, rowsb, accb, sem):
    c = lax.axis_index("c")
    s = lax.axis_index("s")
    w = s * NC + c

    def _accum(table, first):
        if first:
            pltpu.async_copy(table.at[idxb], accb, sem).wait()
        else:
            pltpu.async_copy(table.at[idxb], rowsb, sem).wait()

            def _add(r, _):
                accb[r, 0:16] = accb[r, 0:16] + rowsb[r, 0:16]
                accb[r, 16:32] = accb[r, 16:32] + rowsb[r, 16:32]
                return _
            lax.fori_loop(0, 128, _add, None)

    def _quarter(r, _):
        accb[r, 0:16] = accb[r, 0:16] * 0.25
        accb[r, 16:32] = accb[r, 16:32] * 0.25
        return _

    pltpu.sync_copy(uid1.at[pl.ds(w * 128, 128)], idxb)
    for i, t in enumerate((t0, t1, t2, t3)):
        _accum(t, i == 0)
    lax.fori_loop(0, 128, _quarter, None)
    pltpu.sync_copy(accb, ug.at[pl.ds(w * 128, 128)])

    pltpu.sync_copy(iid1.at[pl.ds(w * 128, 128)], idxb)
    for kk in range(8):
        sl = pl.ds(kk * 16, 16)
        idxb[sl] = idxb[sl] + N_USER
    for i, t in enumerate((t0, t1, t2, t3)):
        _accum(t, i == 0)
    lax.fori_loop(0, 128, _quarter, None)
    pltpu.sync_copy(accb, ig.at[pl.ds(w * 128, 128)])


def _gather_mean(t0, t1, t2, t3, uid1, iid1):
    return pl.kernel(
        _gather_mean_body,
        out_type=(jax.ShapeDtypeStruct((B, D), F32),
                  jax.ShapeDtypeStruct((B, D), F32)),
        mesh=_mesh,
        compiler_params=pltpu.CompilerParams(needs_layout_passes=False, use_tc_tiling_on_sc=False),
        scratch_types=[
            pltpu.VMEM((128,), jnp.int32),
            pltpu.VMEM((128, D), F32),
            pltpu.VMEM((128, D), F32),
            pltpu.SemaphoreType.DMA,
        ],
    )(t0, t1, t2, t3, uid1, iid1)


def kernel(aux_info, user_ids, item_ids, g_src, g_dst, g_vals,
           emb_rate, W_genre, W_director, W_actor,
           emb_gender, emb_age, emb_occ, emb_area,
           emb_user_rel, emb_item_rel,
           fc1_W, fc1_b, fc2_W, fc2_b, out_W, out_b):
    wa = jnp.zeros((KA, 160), F32)
    wa = wa.at[0, 0:32].set(emb_rate[1] - emb_rate[0])
    wa = wa.at[1:26, 32:64].set(W_genre.T)
    wa = wa.at[26:2212, 64:96].set(W_director.T)
    wa = wa.at[2212:KA, 96:128].set(W_actor.T[0:KA - 2212])
    wa = wa.at[1:26, 128].set(1.0)
    wa = wa.at[26:2212, 129].set(1.0)
    wa = wa.at[2212:KA, 130].set(1.0)

    wb = jnp.zeros((KB_HI - KB_LO, 64), F32)
    wb = wb.at[:, 0:32].set(W_actor.T[KB_LO - 2212:KB_HI - 2212])
    wb = wb.at[:, 32].set(1.0)

    ws = jnp.zeros((16, 32), F32)
    ws = ws.at[0].set(emb_rate[0])
    ws = ws.at[1].set(emb_gender[0])
    ws = ws.at[2].set(emb_age[0])
    ws = ws.at[3].set(emb_occ[0])
    ws = ws.at[4].set(emb_area[0])
    ws = ws.at[5].set(emb_gender[1] - emb_gender[0])
    ws = ws.at[6].set(emb_age[1] - emb_age[0])
    ws = ws.at[7].set(emb_occ[1] - emb_occ[0])
    ws = ws.at[8].set(emb_area[1] - emb_area[0])
    ws = ws.at[9].set(W_actor.T[10240 - 2212])
    ws = ws.at[10].set(W_actor.T[10241 - 2212])

    feats = _feats(aux_info, wa, wb, ws)

    pad = EP - g_src.shape[0]
    src1 = jnp.pad(g_src, (0, pad))
    dst1 = jnp.pad(g_dst, (0, pad))
    vals1 = jnp.pad(g_vals, (0, pad))

    t0 = jnp.concatenate([emb_user_rel, emb_item_rel], axis=0)
    t1 = _gcn_layer(t0, src1, dst1, vals1)
    t2 = _gcn_layer(t1, src1, dst1, vals1)
    t3 = _gcn_layer(t2, src1, dst1, vals1)

    ug, ig = _gather_mean(t0, t1, t2, t3, user_ids, item_ids)

    return _mlp(feats, ug, ig, fc1_W, fc1_b, fc2_W, fc2_b, out_W, out_b)
